# baseline (speedup 1.0000x reference)
import functools

import jax
import jax.numpy as jnp
from jax import lax
from jax.experimental import pallas as pl
from jax.experimental.pallas import tpu as pltpu
from jax.experimental.pallas import tpu_sc as plsc

N = 10000
E = 320000
D = 128

NC = 2
NS = 16
NW = NC * NS
CH = 128
E_PAD = 327680
CPW = E_PAD // (NW * CH)
QC = 16
N_PAD = 10240
RPT = N_PAD // NS

_mesh = plsc.VectorSubcoreMesh(core_axis_name="c", subcore_axis_name="s")


def _fill_rows(rows, value):
  v16 = jnp.full((16,), value, jnp.float32)

  def fill(i, carry):
    for k in range(D // 16):
      rows[i, pl.ds(k * 16, 16)] = v16
    return carry

  lax.fori_loop(0, CH, fill, 0)


def _zero_accum(sid, rows, agg_sh):
  for r in range(RPT // CH):
    row0 = sid * RPT + r * CH
    pltpu.sync_copy(rows, agg_sh.at[pl.ds(row0, CH)])


def _writeback(sid, cid, agg_sh, out):
  for r in range(RPT // CH):
    row0 = sid * RPT + r * CH
    pltpu.sync_copy(agg_sh.at[pl.ds(row0, CH)],
                    out.at[pl.ds(cid * N_PAD + row0, CH)])


def _agg_quarters(wid, x_hbm, src_hbm, dst_hbm, srcv, dstv, bufs, gsems,
                  ssems, agg_sh):
  def quarter(q, carry):
    pltpu.sync_copy(src_hbm.at[pl.ds(wid * CPW + q * QC, QC)], srcv)
    pltpu.sync_copy(dst_hbm.at[pl.ds(wid * CPW + q * QC, QC)], dstv)

    pltpu.async_copy(x_hbm.at[srcv.at[0]], bufs[0], gsems[0])
    for j in range(QC):
      b = j % 2
      pltpu.make_async_copy(x_hbm.at[srcv.at[j]], bufs[b],
                            gsems[b]).wait()
      pltpu.async_copy(bufs[b], agg_sh.at[dstv.at[j]], ssems[b],
                       add=True)
      if j + 1 < QC:
        if j >= 1:
          pltpu.make_async_copy(bufs[1 - b],
                                agg_sh.at[dstv.at[j - 1]],
                                ssems[1 - b]).wait()
        pltpu.async_copy(x_hbm.at[srcv.at[j + 1]], bufs[1 - b],
                         gsems[1 - b])
    pltpu.make_async_copy(bufs[(QC - 2) % 2],
                          agg_sh.at[dstv.at[QC - 2]],
                          ssems[(QC - 2) % 2]).wait()
    pltpu.make_async_copy(bufs[(QC - 1) % 2],
                          agg_sh.at[dstv.at[QC - 1]],
                          ssems[(QC - 1) % 2]).wait()
    return carry

  lax.fori_loop(0, CPW // QC, quarter, 0)


def _cnt_quarters(wid, dst_hbm, dstv, ones, sem, agg_sh):
  def quarter(q, carry):
    pltpu.sync_copy(dst_hbm.at[pl.ds(wid * CPW + q * QC, QC)], dstv)
    for j in range(QC):
      pltpu.async_copy(ones, agg_sh.at[dstv.at[j]], sem, add=True)
    for j in range(QC):
      pltpu.make_async_copy(ones, agg_sh.at[dstv.at[j]], sem).wait()
    return carry

  lax.fori_loop(0, CPW // QC, quarter, 0)


def _make_sc_layer1():
  scratch = [
      pltpu.VMEM((QC, CH), jnp.int32),
      pltpu.VMEM((QC, CH), jnp.int32),
      pltpu.VMEM((CH, D), jnp.float32),
      pltpu.VMEM((CH, D), jnp.float32),
      pltpu.VMEM_SHARED((N_PAD, D), jnp.float32),
      pltpu.SemaphoreType.DMA,
      pltpu.SemaphoreType.DMA,
      pltpu.SemaphoreType.DMA,
      pltpu.SemaphoreType.DMA,
  ]

  def body(x_hbm, src_hbm, dst_hbm, agg_out, cnt_out, srcv, dstv, rows0,
           rows1, agg_sh, gsem0, gsem1, ssem0, ssem1):
    cid = lax.axis_index("c")
    sid = lax.axis_index("s")
    wid = sid * NC + cid

    _fill_rows(rows0, 0.0)
    _zero_accum(sid, rows0, agg_sh)
    plsc.subcore_barrier()
    _agg_quarters(wid, x_hbm, src_hbm, dst_hbm, srcv, dstv,
                  (rows0, rows1), (gsem0, gsem1), (ssem0, ssem1), agg_sh)
    plsc.subcore_barrier()
    _writeback(sid, cid, agg_sh, agg_out)

    _fill_rows(rows0, 0.0)
    _zero_accum(sid, rows0, agg_sh)
    _fill_rows(rows0, 1.0)
    plsc.subcore_barrier()
    _cnt_quarters(wid, dst_hbm, dstv, rows0, gsem0, agg_sh)
    plsc.subcore_barrier()
    _writeback(sid, cid, agg_sh, cnt_out)

  return pl.kernel(
      body,
      out_type=(jax.ShapeDtypeStruct((NC * N_PAD, D), jnp.float32),
                jax.ShapeDtypeStruct((NC * N_PAD, D), jnp.float32)),
      mesh=_mesh, scratch_types=scratch)


def _make_sc_agg():
  scratch = [
      pltpu.VMEM((QC, CH), jnp.int32),
      pltpu.VMEM((QC, CH), jnp.int32),
      pltpu.VMEM((CH, D), jnp.float32),
      pltpu.VMEM((CH, D), jnp.float32),
      pltpu.VMEM_SHARED((N_PAD, D), jnp.float32),
      pltpu.SemaphoreType.DMA,
      pltpu.SemaphoreType.DMA,
      pltpu.SemaphoreType.DMA,
      pltpu.SemaphoreType.DMA,
  ]

  def body(x_hbm, src_hbm, dst_hbm, agg_out, srcv, dstv, rows0, rows1,
           agg_sh, gsem0, gsem1, ssem0, ssem1):
    cid = lax.axis_index("c")
    sid = lax.axis_index("s")
    wid = sid * NC + cid

    _fill_rows(rows0, 0.0)
    _zero_accum(sid, rows0, agg_sh)
    plsc.subcore_barrier()
    _agg_quarters(wid, x_hbm, src_hbm, dst_hbm, srcv, dstv,
                  (rows0, rows1), (gsem0, gsem1), (ssem0, ssem1), agg_sh)
    plsc.subcore_barrier()
    _writeback(sid, cid, agg_sh, agg_out)

  return pl.kernel(
      body,
      out_type=jax.ShapeDtypeStruct((NC * N_PAD, D), jnp.float32),
      mesh=_mesh, scratch_types=scratch)


_sc_layer1 = _make_sc_layer1()
_sc_agg = _make_sc_agg()

BR = 1000


def _tc_layer_body(x_ref, agg_ref, cnt_ref, w_ref, b_ref, o_ref):
  cnt = cnt_ref[0, :, 0] + cnt_ref[1, :, 0]
  inv = 1.0 / jnp.maximum(cnt, 1.0)
  agg = agg_ref[0] + agg_ref[1]
  comb = x_ref[...] + agg * inv[:, None]
  h = lax.dot_general(comb, w_ref[...], (((1,), (1,)), ((), ())),
                      preferred_element_type=jnp.float32)
  o_ref[...] = jnp.maximum(h + b_ref[...], 0.0)


def _tc_final_body(x_ref, agg_ref, cnt_ref, w_ref, b_ref, wp_ref, bp_ref,
                   o_ref):
  cnt = cnt_ref[0, :, 0] + cnt_ref[1, :, 0]
  inv = 1.0 / jnp.maximum(cnt, 1.0)
  agg = agg_ref[0] + agg_ref[1]
  comb = x_ref[...] + agg * inv[:, None]
  h = lax.dot_general(comb, w_ref[...], (((1,), (1,)), ((), ())),
                      preferred_element_type=jnp.float32)
  h = jnp.maximum(h + b_ref[...], 0.0)
  p = lax.dot_general(h, wp_ref[...], (((1,), (1,)), ((), ())),
                      preferred_element_type=jnp.float32)
  o_ref[...] = p + bp_ref[...]


_row_spec = pl.BlockSpec((BR, D), lambda i: (i, 0))
_agg_spec = pl.BlockSpec((NC, BR, D), lambda i: (0, i, 0))
_w_spec = pl.BlockSpec((D, D), lambda i: (0, 0))
_b_spec = pl.BlockSpec((1, D), lambda i: (0, 0))

_tc_layer = pl.pallas_call(
    _tc_layer_body,
    grid=(N // BR,),
    in_specs=[_row_spec, _agg_spec, _agg_spec, _w_spec, _b_spec],
    out_specs=_row_spec,
    out_shape=jax.ShapeDtypeStruct((N, D), jnp.float32),
)

_tc_final = pl.pallas_call(
    _tc_final_body,
    grid=(N // BR,),
    in_specs=[_row_spec, _agg_spec, _agg_spec, _w_spec, _b_spec, _w_spec,
              _b_spec],
    out_specs=_row_spec,
    out_shape=jax.ShapeDtypeStruct((N, D), jnp.float32),
)


def kernel(x, edges, W1, b1, W2, b2, Wp, bp):
  src = edges[0]
  dst = edges[1]
  k = jnp.arange(E_PAD - E, dtype=jnp.int32) % (N_PAD - N)
  src_p = jnp.concatenate([src, k]).reshape(NW * CPW, CH)
  dst_p = jnp.concatenate([dst, N + k]).reshape(NW * CPW, CH)

  agg1, cnt1 = _sc_layer1(x, src_p, dst_p)
  agg1 = agg1.reshape(NC, N_PAD, D)
  cnt1 = cnt1.reshape(NC, N_PAD, D)
  h1 = _tc_layer(x, agg1, cnt1, W1, b1.reshape(1, D))
  agg2 = _sc_agg(h1, src_p, dst_p).reshape(NC, N_PAD, D)
  return _tc_final(h1, agg2, cnt1, W2, b2.reshape(1, D), Wp,
                   bp.reshape(1, D))


# speedup vs baseline: 1.0333x; 1.0333x over previous
import functools

import jax
import jax.numpy as jnp
from jax import lax
from jax.experimental import pallas as pl
from jax.experimental.pallas import tpu as pltpu
from jax.experimental.pallas import tpu_sc as plsc

N = 10000
E = 320000
D = 128

NC = 2
NS = 16
NW = NC * NS
CH = 128
E_PAD = 327680
CPW = E_PAD // (NW * CH)
QC = 40
N_PAD = 10240
RPT = N_PAD // NS

_mesh = plsc.VectorSubcoreMesh(core_axis_name="c", subcore_axis_name="s")


def _fill_rows(rows, value):
  v16 = jnp.full((16,), value, jnp.float32)

  def fill(i, carry):
    for k in range(D // 16):
      rows[i, pl.ds(k * 16, 16)] = v16
    return carry

  lax.fori_loop(0, CH, fill, 0)


def _zero_accum(sid, rows, agg_sh):
  for r in range(RPT // CH):
    row0 = sid * RPT + r * CH
    pltpu.sync_copy(rows, agg_sh.at[pl.ds(row0, CH)])


def _writeback(sid, cid, agg_sh, out):
  for r in range(RPT // CH):
    row0 = sid * RPT + r * CH
    pltpu.sync_copy(agg_sh.at[pl.ds(row0, CH)],
                    out.at[pl.ds(cid * N_PAD + row0, CH)])


def _agg_quarters(wid, x_hbm, src_hbm, dst_hbm, srcv, dstv, bufs, gsems,
                  ssems, agg_sh):
  def quarter(q, carry):
    pltpu.sync_copy(src_hbm.at[pl.ds(wid * CPW + q * QC, QC)], srcv)
    pltpu.sync_copy(dst_hbm.at[pl.ds(wid * CPW + q * QC, QC)], dstv)

    pltpu.async_copy(x_hbm.at[srcv.at[0]], bufs[0], gsems[0])
    for j in range(QC):
      b = j % 2
      pltpu.make_async_copy(x_hbm.at[srcv.at[j]], bufs[b],
                            gsems[b]).wait()
      pltpu.async_copy(bufs[b], agg_sh.at[dstv.at[j]], ssems[b],
                       add=True)
      if j + 1 < QC:
        if j >= 1:
          pltpu.make_async_copy(bufs[1 - b],
                                agg_sh.at[dstv.at[j - 1]],
                                ssems[1 - b]).wait()
        pltpu.async_copy(x_hbm.at[srcv.at[j + 1]], bufs[1 - b],
                         gsems[1 - b])
    pltpu.make_async_copy(bufs[(QC - 2) % 2],
                          agg_sh.at[dstv.at[QC - 2]],
                          ssems[(QC - 2) % 2]).wait()
    pltpu.make_async_copy(bufs[(QC - 1) % 2],
                          agg_sh.at[dstv.at[QC - 1]],
                          ssems[(QC - 1) % 2]).wait()
    return carry

  lax.fori_loop(0, CPW // QC, quarter, 0)


def _cnt_quarters(wid, dst_hbm, dstv, ones, sem, agg_sh):
  def quarter(q, carry):
    pltpu.sync_copy(dst_hbm.at[pl.ds(wid * CPW + q * QC, QC)], dstv)
    for j in range(QC):
      pltpu.async_copy(ones, agg_sh.at[dstv.at[j]], sem, add=True)
    for j in range(QC):
      pltpu.make_async_copy(ones, agg_sh.at[dstv.at[j]], sem).wait()
    return carry

  lax.fori_loop(0, CPW // QC, quarter, 0)


def _make_sc_layer1():
  scratch = [
      pltpu.VMEM((QC, CH), jnp.int32),
      pltpu.VMEM((QC, CH), jnp.int32),
      pltpu.VMEM((CH, D), jnp.float32),
      pltpu.VMEM((CH, D), jnp.float32),
      pltpu.VMEM_SHARED((N_PAD, D), jnp.float32),
      pltpu.SemaphoreType.DMA,
      pltpu.SemaphoreType.DMA,
      pltpu.SemaphoreType.DMA,
      pltpu.SemaphoreType.DMA,
  ]

  def body(x_hbm, src_hbm, dst_hbm, agg_out, cnt_out, srcv, dstv, rows0,
           rows1, agg_sh, gsem0, gsem1, ssem0, ssem1):
    cid = lax.axis_index("c")
    sid = lax.axis_index("s")
    wid = sid * NC + cid

    _fill_rows(rows0, 0.0)
    _zero_accum(sid, rows0, agg_sh)
    plsc.subcore_barrier()
    _agg_quarters(wid, x_hbm, src_hbm, dst_hbm, srcv, dstv,
                  (rows0, rows1), (gsem0, gsem1), (ssem0, ssem1), agg_sh)
    plsc.subcore_barrier()
    _writeback(sid, cid, agg_sh, agg_out)

    _fill_rows(rows0, 0.0)
    _zero_accum(sid, rows0, agg_sh)
    _fill_rows(rows0, 1.0)
    plsc.subcore_barrier()
    _cnt_quarters(wid, dst_hbm, dstv, rows0, gsem0, agg_sh)
    plsc.subcore_barrier()
    _writeback(sid, cid, agg_sh, cnt_out)

  return pl.kernel(
      body,
      out_type=(jax.ShapeDtypeStruct((NC * N_PAD, D), jnp.float32),
                jax.ShapeDtypeStruct((NC * N_PAD, D), jnp.float32)),
      mesh=_mesh, scratch_types=scratch)


def _make_sc_agg():
  scratch = [
      pltpu.VMEM((QC, CH), jnp.int32),
      pltpu.VMEM((QC, CH), jnp.int32),
      pltpu.VMEM((CH, D), jnp.float32),
      pltpu.VMEM((CH, D), jnp.float32),
      pltpu.VMEM_SHARED((N_PAD, D), jnp.float32),
      pltpu.SemaphoreType.DMA,
      pltpu.SemaphoreType.DMA,
      pltpu.SemaphoreType.DMA,
      pltpu.SemaphoreType.DMA,
  ]

  def body(x_hbm, src_hbm, dst_hbm, agg_out, srcv, dstv, rows0, rows1,
           agg_sh, gsem0, gsem1, ssem0, ssem1):
    cid = lax.axis_index("c")
    sid = lax.axis_index("s")
    wid = sid * NC + cid

    _fill_rows(rows0, 0.0)
    _zero_accum(sid, rows0, agg_sh)
    plsc.subcore_barrier()
    _agg_quarters(wid, x_hbm, src_hbm, dst_hbm, srcv, dstv,
                  (rows0, rows1), (gsem0, gsem1), (ssem0, ssem1), agg_sh)
    plsc.subcore_barrier()
    _writeback(sid, cid, agg_sh, agg_out)

  return pl.kernel(
      body,
      out_type=jax.ShapeDtypeStruct((NC * N_PAD, D), jnp.float32),
      mesh=_mesh, scratch_types=scratch)


_sc_layer1 = _make_sc_layer1()
_sc_agg = _make_sc_agg()

BR = 1000


def _tc_layer_body(x_ref, agg_ref, cnt_ref, w_ref, b_ref, o_ref):
  cnt = cnt_ref[0, :, 0] + cnt_ref[1, :, 0]
  inv = 1.0 / jnp.maximum(cnt, 1.0)
  agg = agg_ref[0] + agg_ref[1]
  comb = x_ref[...] + agg * inv[:, None]
  h = lax.dot_general(comb, w_ref[...], (((1,), (1,)), ((), ())),
                      preferred_element_type=jnp.float32)
  o_ref[...] = jnp.maximum(h + b_ref[...], 0.0)


def _tc_final_body(x_ref, agg_ref, cnt_ref, w_ref, b_ref, wp_ref, bp_ref,
                   o_ref):
  cnt = cnt_ref[0, :, 0] + cnt_ref[1, :, 0]
  inv = 1.0 / jnp.maximum(cnt, 1.0)
  agg = agg_ref[0] + agg_ref[1]
  comb = x_ref[...] + agg * inv[:, None]
  h = lax.dot_general(comb, w_ref[...], (((1,), (1,)), ((), ())),
                      preferred_element_type=jnp.float32)
  h = jnp.maximum(h + b_ref[...], 0.0)
  p = lax.dot_general(h, wp_ref[...], (((1,), (1,)), ((), ())),
                      preferred_element_type=jnp.float32)
  o_ref[...] = p + bp_ref[...]


_row_spec = pl.BlockSpec((BR, D), lambda i: (i, 0))
_agg_spec = pl.BlockSpec((NC, BR, D), lambda i: (0, i, 0))
_w_spec = pl.BlockSpec((D, D), lambda i: (0, 0))
_b_spec = pl.BlockSpec((1, D), lambda i: (0, 0))

_tc_layer = pl.pallas_call(
    _tc_layer_body,
    grid=(N // BR,),
    in_specs=[_row_spec, _agg_spec, _agg_spec, _w_spec, _b_spec],
    out_specs=_row_spec,
    out_shape=jax.ShapeDtypeStruct((N, D), jnp.float32),
)

_tc_final = pl.pallas_call(
    _tc_final_body,
    grid=(N // BR,),
    in_specs=[_row_spec, _agg_spec, _agg_spec, _w_spec, _b_spec, _w_spec,
              _b_spec],
    out_specs=_row_spec,
    out_shape=jax.ShapeDtypeStruct((N, D), jnp.float32),
)


def kernel(x, edges, W1, b1, W2, b2, Wp, bp):
  src = edges[0]
  dst = edges[1]
  k = jnp.arange(E_PAD - E, dtype=jnp.int32) % (N_PAD - N)
  src_p = jnp.concatenate([src, k]).reshape(NW * CPW, CH)
  dst_p = jnp.concatenate([dst, N + k]).reshape(NW * CPW, CH)

  agg1, cnt1 = _sc_layer1(x, src_p, dst_p)
  agg1 = agg1.reshape(NC, N_PAD, D)
  cnt1 = cnt1.reshape(NC, N_PAD, D)
  h1 = _tc_layer(x, agg1, cnt1, W1, b1.reshape(1, D))
  agg2 = _sc_agg(h1, src_p, dst_p).reshape(NC, N_PAD, D)
  return _tc_final(h1, agg2, cnt1, W2, b2.reshape(1, D), Wp,
                   bp.reshape(1, D))


# speedup vs baseline: 1.1053x; 1.0696x over previous
import functools

import jax
import jax.numpy as jnp
from jax import lax
from jax.experimental import pallas as pl
from jax.experimental.pallas import tpu as pltpu
from jax.experimental.pallas import tpu_sc as plsc

N = 10000
E = 320000
D = 128

NC = 2
NS = 16
NW = NC * NS
CH = 64
E_PAD = 327680
CPW = E_PAD // (NW * CH)
QC = 40
NBUF = 4
N_PAD = 10240
RPT = N_PAD // NS

_mesh = plsc.VectorSubcoreMesh(core_axis_name="c", subcore_axis_name="s")


def _fill_rows(rows, value):
  v16 = jnp.full((16,), value, jnp.float32)

  def fill(i, carry):
    for k in range(D // 16):
      rows[i, pl.ds(k * 16, 16)] = v16
    return carry

  lax.fori_loop(0, CH, fill, 0)


def _zero_accum(sid, rows, agg_sh):
  for r in range(RPT // CH):
    row0 = sid * RPT + r * CH
    pltpu.sync_copy(rows, agg_sh.at[pl.ds(row0, CH)])


def _writeback(sid, cid, agg_sh, out):
  for r in range(RPT // CH):
    row0 = sid * RPT + r * CH
    pltpu.sync_copy(agg_sh.at[pl.ds(row0, CH)],
                    out.at[pl.ds(cid * N_PAD + row0, CH)])


def _agg_quarters(wid, x_hbm, src_hbm, dst_hbm, srcv, dstv, bufs, gsems,
                  ssems, agg_sh):
  def quarter(q, carry):
    pltpu.sync_copy(src_hbm.at[pl.ds(wid * CPW + q * QC, QC)], srcv)
    pltpu.sync_copy(dst_hbm.at[pl.ds(wid * CPW + q * QC, QC)], dstv)

    for p in range(NBUF - 1):
      pltpu.async_copy(x_hbm.at[srcv.at[p]], bufs[p], gsems[p])
    for j in range(QC):
      b = j % NBUF
      pltpu.make_async_copy(x_hbm.at[srcv.at[j]], bufs[b],
                            gsems[b]).wait()
      pltpu.async_copy(bufs[b], agg_sh.at[dstv.at[j]], ssems[b],
                       add=True)
      nxt = j + NBUF - 1
      if nxt < QC:
        nb = nxt % NBUF
        if j >= 1:
          pltpu.make_async_copy(bufs[nb], agg_sh.at[dstv.at[j - 1]],
                                ssems[nb]).wait()
        pltpu.async_copy(x_hbm.at[srcv.at[nxt]], bufs[nb], gsems[nb])
    for j in range(QC - NBUF + 1, QC):
      pltpu.make_async_copy(bufs[j % NBUF], agg_sh.at[dstv.at[j]],
                            ssems[j % NBUF]).wait()
    return carry

  lax.fori_loop(0, CPW // QC, quarter, 0)


def _cnt_quarters(wid, dst_hbm, dstv, ones, sem, agg_sh):
  def quarter(q, carry):
    pltpu.sync_copy(dst_hbm.at[pl.ds(wid * CPW + q * QC, QC)], dstv)
    for j in range(QC):
      pltpu.async_copy(ones, agg_sh.at[dstv.at[j]], sem, add=True)
    for j in range(QC):
      pltpu.make_async_copy(ones, agg_sh.at[dstv.at[j]], sem).wait()
    return carry

  lax.fori_loop(0, CPW // QC, quarter, 0)


def _make_sc_layer1():
  scratch = (
      [pltpu.VMEM((QC, CH), jnp.int32)] * 2
      + [pltpu.VMEM((CH, D), jnp.float32)] * NBUF
      + [pltpu.VMEM_SHARED((N_PAD, D), jnp.float32)]
      + [pltpu.SemaphoreType.DMA] * (2 * NBUF)
  )

  def body(x_hbm, src_hbm, dst_hbm, agg_out, cnt_out, srcv, dstv, *rest):
    bufs = rest[:NBUF]
    agg_sh = rest[NBUF]
    gsems = rest[NBUF + 1:2 * NBUF + 1]
    ssems = rest[2 * NBUF + 1:]
    rows0 = bufs[0]
    cid = lax.axis_index("c")
    sid = lax.axis_index("s")
    wid = sid * NC + cid

    _fill_rows(rows0, 0.0)
    _zero_accum(sid, rows0, agg_sh)
    plsc.subcore_barrier()
    _agg_quarters(wid, x_hbm, src_hbm, dst_hbm, srcv, dstv,
                  bufs, gsems, ssems, agg_sh)
    plsc.subcore_barrier()
    _writeback(sid, cid, agg_sh, agg_out)

    _fill_rows(rows0, 0.0)
    _zero_accum(sid, rows0, agg_sh)
    _fill_rows(rows0, 1.0)
    plsc.subcore_barrier()
    _cnt_quarters(wid, dst_hbm, dstv, rows0, gsems[0], agg_sh)
    plsc.subcore_barrier()
    _writeback(sid, cid, agg_sh, cnt_out)

  return pl.kernel(
      body,
      out_type=(jax.ShapeDtypeStruct((NC * N_PAD, D), jnp.float32),
                jax.ShapeDtypeStruct((NC * N_PAD, D), jnp.float32)),
      mesh=_mesh, scratch_types=scratch)


def _make_sc_agg():
  scratch = (
      [pltpu.VMEM((QC, CH), jnp.int32)] * 2
      + [pltpu.VMEM((CH, D), jnp.float32)] * NBUF
      + [pltpu.VMEM_SHARED((N_PAD, D), jnp.float32)]
      + [pltpu.SemaphoreType.DMA] * (2 * NBUF)
  )

  def body(x_hbm, src_hbm, dst_hbm, agg_out, srcv, dstv, *rest):
    bufs = rest[:NBUF]
    agg_sh = rest[NBUF]
    gsems = rest[NBUF + 1:2 * NBUF + 1]
    ssems = rest[2 * NBUF + 1:]
    cid = lax.axis_index("c")
    sid = lax.axis_index("s")
    wid = sid * NC + cid

    _fill_rows(bufs[0], 0.0)
    _zero_accum(sid, bufs[0], agg_sh)
    plsc.subcore_barrier()
    _agg_quarters(wid, x_hbm, src_hbm, dst_hbm, srcv, dstv,
                  bufs, gsems, ssems, agg_sh)
    plsc.subcore_barrier()
    _writeback(sid, cid, agg_sh, agg_out)

  return pl.kernel(
      body,
      out_type=jax.ShapeDtypeStruct((NC * N_PAD, D), jnp.float32),
      mesh=_mesh, scratch_types=scratch)


_sc_layer1 = _make_sc_layer1()
_sc_agg = _make_sc_agg()

BR = 1000


def _tc_layer_body(x_ref, agg_ref, cnt_ref, w_ref, b_ref, o_ref):
  cnt = cnt_ref[0, :, 0] + cnt_ref[1, :, 0]
  inv = 1.0 / jnp.maximum(cnt, 1.0)
  agg = agg_ref[0] + agg_ref[1]
  comb = x_ref[...] + agg * inv[:, None]
  h = lax.dot_general(comb, w_ref[...], (((1,), (1,)), ((), ())),
                      preferred_element_type=jnp.float32)
  o_ref[...] = jnp.maximum(h + b_ref[...], 0.0)


def _tc_final_body(x_ref, agg_ref, cnt_ref, w_ref, b_ref, wp_ref, bp_ref,
                   o_ref):
  cnt = cnt_ref[0, :, 0] + cnt_ref[1, :, 0]
  inv = 1.0 / jnp.maximum(cnt, 1.0)
  agg = agg_ref[0] + agg_ref[1]
  comb = x_ref[...] + agg * inv[:, None]
  h = lax.dot_general(comb, w_ref[...], (((1,), (1,)), ((), ())),
                      preferred_element_type=jnp.float32)
  h = jnp.maximum(h + b_ref[...], 0.0)
  p = lax.dot_general(h, wp_ref[...], (((1,), (1,)), ((), ())),
                      preferred_element_type=jnp.float32)
  o_ref[...] = p + bp_ref[...]


_row_spec = pl.BlockSpec((BR, D), lambda i: (i, 0))
_agg_spec = pl.BlockSpec((NC, BR, D), lambda i: (0, i, 0))
_w_spec = pl.BlockSpec((D, D), lambda i: (0, 0))
_b_spec = pl.BlockSpec((1, D), lambda i: (0, 0))

_tc_layer = pl.pallas_call(
    _tc_layer_body,
    grid=(N // BR,),
    in_specs=[_row_spec, _agg_spec, _agg_spec, _w_spec, _b_spec],
    out_specs=_row_spec,
    out_shape=jax.ShapeDtypeStruct((N, D), jnp.float32),
)

_tc_final = pl.pallas_call(
    _tc_final_body,
    grid=(N // BR,),
    in_specs=[_row_spec, _agg_spec, _agg_spec, _w_spec, _b_spec, _w_spec,
              _b_spec],
    out_specs=_row_spec,
    out_shape=jax.ShapeDtypeStruct((N, D), jnp.float32),
)


def kernel(x, edges, W1, b1, W2, b2, Wp, bp):
  src = edges[0]
  dst = edges[1]
  k = jnp.arange(E_PAD - E, dtype=jnp.int32) % (N_PAD - N)
  src_p = jnp.concatenate([src, k]).reshape(NW * CPW, CH)
  dst_p = jnp.concatenate([dst, N + k]).reshape(NW * CPW, CH)

  agg1, cnt1 = _sc_layer1(x, src_p, dst_p)
  agg1 = agg1.reshape(NC, N_PAD, D)
  cnt1 = cnt1.reshape(NC, N_PAD, D)
  h1 = _tc_layer(x, agg1, cnt1, W1, b1.reshape(1, D))
  agg2 = _sc_agg(h1, src_p, dst_p).reshape(NC, N_PAD, D)
  return _tc_final(h1, agg2, cnt1, W2, b2.reshape(1, D), Wp,
                   bp.reshape(1, D))


# speedup vs baseline: 1.1156x; 1.0093x over previous
import functools

import jax
import jax.numpy as jnp
from jax import lax
from jax.experimental import pallas as pl
from jax.experimental.pallas import tpu as pltpu
from jax.experimental.pallas import tpu_sc as plsc

N = 10000
E = 320000
D = 128

NC = 2
NS = 16
NW = NC * NS
CH = 64
E_PAD = 327680
CPW = E_PAD // (NW * CH)
QC = 40
NBUF = 4
N_PAD = 10240
RPT = N_PAD // NS

_mesh = plsc.VectorSubcoreMesh(core_axis_name="c", subcore_axis_name="s")


def _fill_rows(rows, value):
  v16 = jnp.full((16,), value, jnp.float32)

  def fill(i, carry):
    for k in range(D // 16):
      rows[i, pl.ds(k * 16, 16)] = v16
    return carry

  lax.fori_loop(0, CH, fill, 0)


def _zero_accum(sid, rows, agg_sh):
  for r in range(RPT // CH):
    row0 = sid * RPT + r * CH
    pltpu.sync_copy(rows, agg_sh.at[pl.ds(row0, CH)])


def _writeback(sid, cid, agg_sh, out):
  for r in range(RPT // CH):
    row0 = sid * RPT + r * CH
    pltpu.sync_copy(agg_sh.at[pl.ds(row0, CH)],
                    out.at[pl.ds(cid * N_PAD + row0, CH)])


def _agg_quarters(wid, x_hbm, src_hbm, dst_hbm, srcv, dstv, bufs, gsems,
                  ssems, agg_sh):
  def quarter(q, carry):
    pltpu.sync_copy(src_hbm.at[pl.ds(wid * CPW + q * QC, QC)], srcv)
    pltpu.sync_copy(dst_hbm.at[pl.ds(wid * CPW + q * QC, QC)], dstv)

    for p in range(NBUF - 1):
      pltpu.async_copy(x_hbm.at[srcv.at[p]], bufs[p], gsems[p])
    for j in range(QC):
      b = j % NBUF
      pltpu.make_async_copy(x_hbm.at[srcv.at[j]], bufs[b],
                            gsems[b]).wait()
      pltpu.async_copy(bufs[b], agg_sh.at[dstv.at[j]], ssems[b],
                       add=True)
      nxt = j + NBUF - 1
      if nxt < QC:
        nb = nxt % NBUF
        if j >= 1:
          pltpu.make_async_copy(bufs[nb], agg_sh.at[dstv.at[j - 1]],
                                ssems[nb]).wait()
        pltpu.async_copy(x_hbm.at[srcv.at[nxt]], bufs[nb], gsems[nb])
    for j in range(QC - NBUF, QC):
      pltpu.make_async_copy(bufs[j % NBUF], agg_sh.at[dstv.at[j]],
                            ssems[j % NBUF]).wait()
    return carry

  lax.fori_loop(0, CPW // QC, quarter, 0)


def _cnt_quarters(wid, dst_hbm, dstv, ones, sem, agg_sh):
  def quarter(q, carry):
    pltpu.sync_copy(dst_hbm.at[pl.ds(wid * CPW + q * QC, QC)], dstv)
    for j in range(QC):
      pltpu.async_copy(ones, agg_sh.at[dstv.at[j]], sem, add=True)
    for j in range(QC):
      pltpu.make_async_copy(ones, agg_sh.at[dstv.at[j]], sem).wait()
    return carry

  lax.fori_loop(0, CPW // QC, quarter, 0)


def _make_sc_layer1():
  scratch = (
      [pltpu.VMEM((QC, CH), jnp.int32)] * 2
      + [pltpu.VMEM((CH, D), jnp.float32)] * NBUF
      + [pltpu.VMEM_SHARED((N_PAD, D), jnp.float32)]
      + [pltpu.SemaphoreType.DMA] * (2 * NBUF)
  )

  def body(x_hbm, src_hbm, dst_hbm, agg_out, cnt_out, srcv, dstv, *rest):
    bufs = rest[:NBUF]
    agg_sh = rest[NBUF]
    gsems = rest[NBUF + 1:2 * NBUF + 1]
    ssems = rest[2 * NBUF + 1:]
    rows0 = bufs[0]
    cid = lax.axis_index("c")
    sid = lax.axis_index("s")
    wid = sid * NC + cid

    _fill_rows(rows0, 0.0)
    _zero_accum(sid, rows0, agg_sh)
    plsc.subcore_barrier()
    _agg_quarters(wid, x_hbm, src_hbm, dst_hbm, srcv, dstv,
                  bufs, gsems, ssems, agg_sh)
    plsc.subcore_barrier()
    _writeback(sid, cid, agg_sh, agg_out)

    _fill_rows(rows0, 0.0)
    _zero_accum(sid, rows0, agg_sh)
    _fill_rows(rows0, 1.0)
    plsc.subcore_barrier()
    _cnt_quarters(wid, dst_hbm, dstv, rows0, gsems[0], agg_sh)
    plsc.subcore_barrier()
    _writeback(sid, cid, agg_sh, cnt_out)

  return pl.kernel(
      body,
      out_type=(jax.ShapeDtypeStruct((NC * N_PAD, D), jnp.float32),
                jax.ShapeDtypeStruct((NC * N_PAD, D), jnp.float32)),
      mesh=_mesh, scratch_types=scratch)


def _make_sc_agg():
  scratch = (
      [pltpu.VMEM((QC, CH), jnp.int32)] * 2
      + [pltpu.VMEM((CH, D), jnp.float32)] * NBUF
      + [pltpu.VMEM_SHARED((N_PAD, D), jnp.float32)]
      + [pltpu.SemaphoreType.DMA] * (2 * NBUF)
  )

  def body(x_hbm, src_hbm, dst_hbm, agg_out, srcv, dstv, *rest):
    bufs = rest[:NBUF]
    agg_sh = rest[NBUF]
    gsems = rest[NBUF + 1:2 * NBUF + 1]
    ssems = rest[2 * NBUF + 1:]
    cid = lax.axis_index("c")
    sid = lax.axis_index("s")
    wid = sid * NC + cid

    _fill_rows(bufs[0], 0.0)
    _zero_accum(sid, bufs[0], agg_sh)
    plsc.subcore_barrier()
    _agg_quarters(wid, x_hbm, src_hbm, dst_hbm, srcv, dstv,
                  bufs, gsems, ssems, agg_sh)
    plsc.subcore_barrier()
    _writeback(sid, cid, agg_sh, agg_out)

  return pl.kernel(
      body,
      out_type=jax.ShapeDtypeStruct((NC * N_PAD, D), jnp.float32),
      mesh=_mesh, scratch_types=scratch)


_sc_layer1 = _make_sc_layer1()
_sc_agg = _make_sc_agg()

BR = 1000


def _tc_layer_body(x_ref, agg_ref, cnt_ref, w_ref, b_ref, o_ref):
  cnt = cnt_ref[0, :, 0] + cnt_ref[1, :, 0]
  inv = 1.0 / jnp.maximum(cnt, 1.0)
  agg = agg_ref[0] + agg_ref[1]
  comb = x_ref[...] + agg * inv[:, None]
  h = lax.dot_general(comb, w_ref[...], (((1,), (1,)), ((), ())),
                      preferred_element_type=jnp.float32)
  o_ref[...] = jnp.maximum(h + b_ref[...], 0.0)


def _tc_final_body(x_ref, agg_ref, cnt_ref, w_ref, b_ref, wp_ref, bp_ref,
                   o_ref):
  cnt = cnt_ref[0, :, 0] + cnt_ref[1, :, 0]
  inv = 1.0 / jnp.maximum(cnt, 1.0)
  agg = agg_ref[0] + agg_ref[1]
  comb = x_ref[...] + agg * inv[:, None]
  h = lax.dot_general(comb, w_ref[...], (((1,), (1,)), ((), ())),
                      preferred_element_type=jnp.float32)
  h = jnp.maximum(h + b_ref[...], 0.0)
  p = lax.dot_general(h, wp_ref[...], (((1,), (1,)), ((), ())),
                      preferred_element_type=jnp.float32)
  o_ref[...] = p + bp_ref[...]


_row_spec = pl.BlockSpec((BR, D), lambda i: (i, 0))
_agg_spec = pl.BlockSpec((NC, BR, D), lambda i: (0, i, 0))
_w_spec = pl.BlockSpec((D, D), lambda i: (0, 0))
_b_spec = pl.BlockSpec((1, D), lambda i: (0, 0))

_tc_layer = pl.pallas_call(
    _tc_layer_body,
    grid=(N // BR,),
    in_specs=[_row_spec, _agg_spec, _agg_spec, _w_spec, _b_spec],
    out_specs=_row_spec,
    out_shape=jax.ShapeDtypeStruct((N, D), jnp.float32),
)

_tc_final = pl.pallas_call(
    _tc_final_body,
    grid=(N // BR,),
    in_specs=[_row_spec, _agg_spec, _agg_spec, _w_spec, _b_spec, _w_spec,
              _b_spec],
    out_specs=_row_spec,
    out_shape=jax.ShapeDtypeStruct((N, D), jnp.float32),
)


def kernel(x, edges, W1, b1, W2, b2, Wp, bp):
  src = edges[0]
  dst = edges[1]
  k = jnp.arange(E_PAD - E, dtype=jnp.int32) % (N_PAD - N)
  src_p = jnp.concatenate([src, k]).reshape(NW * CPW, CH)
  dst_p = jnp.concatenate([dst, N + k]).reshape(NW * CPW, CH)

  agg1, cnt1 = _sc_layer1(x, src_p, dst_p)
  agg1 = agg1.reshape(NC, N_PAD, D)
  cnt1 = cnt1.reshape(NC, N_PAD, D)
  h1 = _tc_layer(x, agg1, cnt1, W1, b1.reshape(1, D))
  agg2 = _sc_agg(h1, src_p, dst_p).reshape(NC, N_PAD, D)
  return _tc_final(h1, agg2, cnt1, W2, b2.reshape(1, D), Wp,
                   bp.reshape(1, D))
